# baseline (speedup 1.0000x reference)
import functools

import jax
import jax.numpy as jnp
import numpy as np
from jax import lax
from jax.experimental import pallas as pl
from jax.experimental.pallas import tpu as pltpu
from jax.experimental.pallas import tpu_sc as plsc

_N = 10000
_D = 128
_E = 320000
_BN_EPS = 1e-5
_BN_SCALE = float(1.0 / np.sqrt(1.0 + _BN_EPS))

_NC = 2
_NS = 16
_NW = _NC * _NS
_CH = 128
_CPT = 79
_EPAD = _NW * _CPT * _CH
_NPAD = 10112
_RPS = _NPAD // _NS
_RLAST = _N - (_NS - 1) * _RPS


def _sc_segment_sum(h, src1, dst1, zrows):
    mesh = plsc.VectorSubcoreMesh(
        core_axis_name="c", subcore_axis_name="s",
        num_cores=_NC, num_subcores=_NS)

    @functools.partial(
        pl.kernel,
        out_type=jax.ShapeDtypeStruct((_NC, _N, _D), jnp.float32),
        mesh=mesh,
        scratch_types=[
            pltpu.VMEM_SHARED((_NPAD, _D), jnp.float32),
            pltpu.VMEM((_CH,), jnp.int32),
            pltpu.VMEM((_CH,), jnp.int32),
            pltpu.VMEM((_CH, _D), jnp.float32),
        ],
    )
    def k(h_hbm, src_hbm, dst_hbm, z_hbm, out_hbm, agg_sh, s0, d0, r0):
        c = lax.axis_index("c")
        s = lax.axis_index("s")
        wid = s * _NC + c
        rbase = s * _RPS
        ebase = wid * (_CPT * _CH)

        pltpu.sync_copy(z_hbm, agg_sh.at[pl.ds(rbase, _RPS)])
        plsc.subcore_barrier()

        @pl.loop(0, _CPT)
        def _(j):
            off = ebase + j * _CH
            pltpu.sync_copy(src_hbm.at[pl.ds(off, _CH)], s0)
            pltpu.sync_copy(dst_hbm.at[pl.ds(off, _CH)], d0)
            pltpu.sync_copy(h_hbm.at[s0], r0)
            pltpu.sync_copy(r0, agg_sh.at[d0], add=True)

        plsc.subcore_barrier()

        @pl.when(s < _NS - 1)
        def _():
            pltpu.sync_copy(agg_sh.at[pl.ds(rbase, _RPS)],
                            out_hbm.at[c, pl.ds(rbase, _RPS)])

        @pl.when(s == _NS - 1)
        def _():
            pltpu.sync_copy(agg_sh.at[pl.ds(rbase, _RLAST)],
                            out_hbm.at[c, pl.ds(rbase, _RLAST)])

    return k(h, src1, dst1, zrows)


def _mlp(h, agg, W1, b1r, W2f, b2f, epsv, relu_out):
    BR = 1000

    def body(eps_ref, h_ref, agg_ref, w1_ref, b1_ref, w2_ref, b2_ref, out_ref):
        z = eps_ref[...] * h_ref[...] + agg_ref[0] + agg_ref[1]
        z = jnp.dot(z, w1_ref[...], preferred_element_type=jnp.float32) + b1_ref[...]
        z = jnp.maximum(z, 0.0)
        z = jnp.dot(z, w2_ref[...], preferred_element_type=jnp.float32) + b2_ref[...]
        if relu_out:
            z = jnp.maximum(z, 0.0)
        out_ref[...] = z

    return pl.pallas_call(
        body,
        grid=(_N // BR,),
        in_specs=[
            pl.BlockSpec((1, _D), lambda i: (0, 0)),
            pl.BlockSpec((BR, _D), lambda i: (i, 0)),
            pl.BlockSpec((_NC, BR, _D), lambda i: (0, i, 0)),
            pl.BlockSpec((_D, _D), lambda i: (0, 0)),
            pl.BlockSpec((1, _D), lambda i: (0, 0)),
            pl.BlockSpec((_D, _D), lambda i: (0, 0)),
            pl.BlockSpec((1, _D), lambda i: (0, 0)),
        ],
        out_specs=pl.BlockSpec((BR, _D), lambda i: (i, 0)),
        out_shape=jax.ShapeDtypeStruct((_N, _D), jnp.float32),
    )(epsv, h, agg, W1, b1r, W2f, b2f)


def kernel(x, edge_index,
           W1_0, b1_0, W2_0, b2_0, eps_0, gamma_0, beta_0,
           W1_1, b1_1, W2_1, b2_1, eps_1, gamma_1, beta_1,
           W1_2, b1_2, W2_2, b2_2, eps_2, gamma_2, beta_2):
    npad_e = _EPAD - _E
    src_pad = jnp.arange(npad_e, dtype=jnp.int32) % _N
    dst_pad = (_N + (jnp.arange(npad_e, dtype=jnp.int32) % (_NPAD - _N)))
    src1 = jnp.concatenate([edge_index[0], src_pad])
    dst1 = jnp.concatenate([edge_index[1], dst_pad])
    zrows = jnp.zeros((_RPS, _D), jnp.float32)

    layers = [
        (W1_0, b1_0, W2_0, b2_0, eps_0, gamma_0, beta_0),
        (W1_1, b1_1, W2_1, b2_1, eps_1, gamma_1, beta_1),
        (W1_2, b1_2, W2_2, b2_2, eps_2, gamma_2, beta_2),
    ]
    h = x
    for i, (W1, b1, W2, b2, eps, gamma, beta) in enumerate(layers):
        agg = _sc_segment_sum(h, src1, dst1, zrows)
        gs = gamma * _BN_SCALE
        W2f = W2 * gs[None, :]
        b2f = (b2 * gs + beta).reshape(1, _D)
        epsv = jnp.broadcast_to(1.0 + eps, (1, _D)).astype(jnp.float32)
        h = _mlp(h, agg, W1, b1.reshape(1, _D), W2f, b2f, epsv, i < 2)
    return h


# speedup vs baseline: 1.0794x; 1.0794x over previous
import functools

import jax
import jax.numpy as jnp
import numpy as np
from jax import lax
from jax.experimental import pallas as pl
from jax.experimental.pallas import tpu as pltpu
from jax.experimental.pallas import tpu_sc as plsc

_N = 10000
_D = 128
_E = 320000
_BN_EPS = 1e-5
_BN_SCALE = float(1.0 / np.sqrt(1.0 + _BN_EPS))

_NC = 2
_NS = 16
_NW = _NC * _NS
_CH = 128
_CPT = 80
_EPAD = _NW * _CPT * _CH
_NPAD = 10112
_RPS = _NPAD // _NS
_RLAST = _N - (_NS - 1) * _RPS


def _sc_segment_sum(h, src1, dst1, zrows):
    mesh = plsc.VectorSubcoreMesh(
        core_axis_name="c", subcore_axis_name="s",
        num_cores=_NC, num_subcores=_NS)

    @functools.partial(
        pl.kernel,
        out_type=jax.ShapeDtypeStruct((_NC, _N, _D), jnp.float32),
        mesh=mesh,
        scratch_types=[
            pltpu.VMEM_SHARED((_NPAD, _D), jnp.float32),
            pltpu.VMEM((_CH,), jnp.int32),
            pltpu.VMEM((_CH,), jnp.int32),
            pltpu.VMEM((_CH,), jnp.int32),
            pltpu.VMEM((_CH,), jnp.int32),
            pltpu.VMEM((_CH, _D), jnp.float32),
            pltpu.VMEM((_CH, _D), jnp.float32),
            pltpu.SemaphoreType.DMA,
            pltpu.SemaphoreType.DMA,
        ],
    )
    def k(h_hbm, src_hbm, dst_hbm, z_hbm, out_hbm, agg_sh,
          s0, s1, d0, d1, r0, r1, m0, m1):
        c = lax.axis_index("c")
        s = lax.axis_index("s")
        wid = s * _NC + c
        rbase = s * _RPS
        ebase = wid * (_CPT * _CH)

        pltpu.sync_copy(z_hbm, agg_sh.at[pl.ds(rbase, _RPS)])
        plsc.subcore_barrier()

        @pl.loop(0, _CPT // 2)
        def _(p):
            off0 = ebase + p * (2 * _CH)
            pltpu.sync_copy(src_hbm.at[pl.ds(off0, _CH)], s0)
            pltpu.sync_copy(dst_hbm.at[pl.ds(off0, _CH)], d0)
            pltpu.sync_copy(src_hbm.at[pl.ds(off0 + _CH, _CH)], s1)
            pltpu.sync_copy(dst_hbm.at[pl.ds(off0 + _CH, _CH)], d1)
            g0 = pltpu.async_copy(h_hbm.at[s0], r0, m0)
            g1 = pltpu.async_copy(h_hbm.at[s1], r1, m1)
            g0.wait()
            sc0 = pltpu.async_copy(r0, agg_sh.at[d0], m0, add=True)
            g1.wait()
            sc1 = pltpu.async_copy(r1, agg_sh.at[d1], m1, add=True)
            sc0.wait()
            sc1.wait()

        plsc.subcore_barrier()

        @pl.when(s < _NS - 1)
        def _():
            pltpu.sync_copy(agg_sh.at[pl.ds(rbase, _RPS)],
                            out_hbm.at[c, pl.ds(rbase, _RPS)])

        @pl.when(s == _NS - 1)
        def _():
            pltpu.sync_copy(agg_sh.at[pl.ds(rbase, _RLAST)],
                            out_hbm.at[c, pl.ds(rbase, _RLAST)])

    return k(h, src1, dst1, zrows)


def _mlp(h, agg, W1, b1r, W2f, b2f, epsv, relu_out):
    BR = 1000

    def body(eps_ref, h_ref, agg_ref, w1_ref, b1_ref, w2_ref, b2_ref, out_ref):
        z = eps_ref[...] * h_ref[...] + agg_ref[0] + agg_ref[1]
        z = jnp.dot(z, w1_ref[...], preferred_element_type=jnp.float32) + b1_ref[...]
        z = jnp.maximum(z, 0.0)
        z = jnp.dot(z, w2_ref[...], preferred_element_type=jnp.float32) + b2_ref[...]
        if relu_out:
            z = jnp.maximum(z, 0.0)
        out_ref[...] = z

    return pl.pallas_call(
        body,
        grid=(_N // BR,),
        in_specs=[
            pl.BlockSpec((1, _D), lambda i: (0, 0)),
            pl.BlockSpec((BR, _D), lambda i: (i, 0)),
            pl.BlockSpec((_NC, BR, _D), lambda i: (0, i, 0)),
            pl.BlockSpec((_D, _D), lambda i: (0, 0)),
            pl.BlockSpec((1, _D), lambda i: (0, 0)),
            pl.BlockSpec((_D, _D), lambda i: (0, 0)),
            pl.BlockSpec((1, _D), lambda i: (0, 0)),
        ],
        out_specs=pl.BlockSpec((BR, _D), lambda i: (i, 0)),
        out_shape=jax.ShapeDtypeStruct((_N, _D), jnp.float32),
    )(epsv, h, agg, W1, b1r, W2f, b2f)


def kernel(x, edge_index,
           W1_0, b1_0, W2_0, b2_0, eps_0, gamma_0, beta_0,
           W1_1, b1_1, W2_1, b2_1, eps_1, gamma_1, beta_1,
           W1_2, b1_2, W2_2, b2_2, eps_2, gamma_2, beta_2):
    npad_e = _EPAD - _E
    src_pad = jnp.arange(npad_e, dtype=jnp.int32) % _N
    dst_pad = (_N + (jnp.arange(npad_e, dtype=jnp.int32) % (_NPAD - _N)))
    src1 = jnp.concatenate([edge_index[0], src_pad])
    dst1 = jnp.concatenate([edge_index[1], dst_pad])
    zrows = jnp.zeros((_RPS, _D), jnp.float32)

    layers = [
        (W1_0, b1_0, W2_0, b2_0, eps_0, gamma_0, beta_0),
        (W1_1, b1_1, W2_1, b2_1, eps_1, gamma_1, beta_1),
        (W1_2, b1_2, W2_2, b2_2, eps_2, gamma_2, beta_2),
    ]
    h = x
    for i, (W1, b1, W2, b2, eps, gamma, beta) in enumerate(layers):
        agg = _sc_segment_sum(h, src1, dst1, zrows)
        gs = gamma * _BN_SCALE
        W2f = W2 * gs[None, :]
        b2f = (b2 * gs + beta).reshape(1, _D)
        epsv = jnp.broadcast_to(1.0 + eps, (1, _D)).astype(jnp.float32)
        h = _mlp(h, agg, W1, b1.reshape(1, _D), W2f, b2f, epsv, i < 2)
    return h


# speedup vs baseline: 1.4123x; 1.3084x over previous
import functools

import jax
import jax.numpy as jnp
import numpy as np
from jax import lax
from jax.experimental import pallas as pl
from jax.experimental.pallas import tpu as pltpu
from jax.experimental.pallas import tpu_sc as plsc

_N = 10000
_D = 128
_E = 320000
_BN_EPS = 1e-5
_BN_SCALE = float(1.0 / np.sqrt(1.0 + _BN_EPS))

_NC = 2
_NS = 16
_NW = _NC * _NS
_CH = 128
_CPT = 80
_EPAD = _NW * _CPT * _CH
_NPAD = 10112
_RPS = _NPAD // _NS
_RLAST = _N - (_NS - 1) * _RPS


def _sc_segment_sum(h, src1, dst1, zrows):
    mesh = plsc.VectorSubcoreMesh(
        core_axis_name="c", subcore_axis_name="s",
        num_cores=_NC, num_subcores=_NS)

    @functools.partial(
        pl.kernel,
        out_type=jax.ShapeDtypeStruct((_NC, _N, _D), jnp.float32),
        mesh=mesh,
        scratch_types=[
            pltpu.VMEM_SHARED((_NPAD, _D), jnp.float32),
            pltpu.VMEM((_CH,), jnp.int32),
            pltpu.VMEM((_CH,), jnp.int32),
            pltpu.VMEM((_CH,), jnp.int32),
            pltpu.VMEM((_CH,), jnp.int32),
            pltpu.VMEM((_CH, _D), jnp.float32),
            pltpu.VMEM((_CH, _D), jnp.float32),
            pltpu.SemaphoreType.DMA,
            pltpu.SemaphoreType.DMA,
        ],
    )
    def k(h_hbm, src_hbm, dst_hbm, z_hbm, out_hbm, agg_sh,
          s0, s1, d0, d1, r0, r1, m0, m1):
        c = lax.axis_index("c")
        s = lax.axis_index("s")
        wid = s * _NC + c
        rbase = s * _RPS
        ebase = wid * (_CPT * _CH)

        pltpu.sync_copy(z_hbm, agg_sh.at[pl.ds(rbase, _RPS)])
        plsc.subcore_barrier()

        @pl.loop(0, _CPT // 2)
        def _(p):
            off0 = ebase + p * (2 * _CH)
            i0a = pltpu.async_copy(src_hbm.at[pl.ds(off0, _CH)], s0, m0)
            i0b = pltpu.async_copy(dst_hbm.at[pl.ds(off0, _CH)], d0, m0)
            i1a = pltpu.async_copy(src_hbm.at[pl.ds(off0 + _CH, _CH)], s1, m1)
            i1b = pltpu.async_copy(dst_hbm.at[pl.ds(off0 + _CH, _CH)], d1, m1)
            i0a.wait()
            i0b.wait()
            g0 = pltpu.async_copy(h_hbm.at[s0], r0, m0)
            i1a.wait()
            i1b.wait()
            g1 = pltpu.async_copy(h_hbm.at[s1], r1, m1)
            g0.wait()
            sc0 = pltpu.async_copy(r0, agg_sh.at[d0], m0, add=True)
            g1.wait()
            sc1 = pltpu.async_copy(r1, agg_sh.at[d1], m1, add=True)
            sc0.wait()
            sc1.wait()

        plsc.subcore_barrier()

        @pl.when(s < _NS - 1)
        def _():
            pltpu.sync_copy(agg_sh.at[pl.ds(rbase, _RPS)],
                            out_hbm.at[c, pl.ds(rbase, _RPS)])

        @pl.when(s == _NS - 1)
        def _():
            pltpu.sync_copy(agg_sh.at[pl.ds(rbase, _RLAST)],
                            out_hbm.at[c, pl.ds(rbase, _RLAST)])

    return k(h, src1, dst1, zrows)


def _mlp(h, agg, W1, b1r, W2f, b2f, epsv, relu_out):
    BR = 1000

    def body(eps_ref, h_ref, agg_ref, w1_ref, b1_ref, w2_ref, b2_ref, out_ref):
        z = eps_ref[...] * h_ref[...] + agg_ref[0] + agg_ref[1]
        z = jnp.dot(z, w1_ref[...], preferred_element_type=jnp.float32) + b1_ref[...]
        z = jnp.maximum(z, 0.0)
        z = jnp.dot(z, w2_ref[...], preferred_element_type=jnp.float32) + b2_ref[...]
        if relu_out:
            z = jnp.maximum(z, 0.0)
        out_ref[...] = z

    return pl.pallas_call(
        body,
        grid=(_N // BR,),
        in_specs=[
            pl.BlockSpec((1, _D), lambda i: (0, 0)),
            pl.BlockSpec((BR, _D), lambda i: (i, 0)),
            pl.BlockSpec((_NC, BR, _D), lambda i: (0, i, 0)),
            pl.BlockSpec((_D, _D), lambda i: (0, 0)),
            pl.BlockSpec((1, _D), lambda i: (0, 0)),
            pl.BlockSpec((_D, _D), lambda i: (0, 0)),
            pl.BlockSpec((1, _D), lambda i: (0, 0)),
        ],
        out_specs=pl.BlockSpec((BR, _D), lambda i: (i, 0)),
        out_shape=jax.ShapeDtypeStruct((_N, _D), jnp.float32),
    )(epsv, h, agg, W1, b1r, W2f, b2f)


def kernel(x, edge_index,
           W1_0, b1_0, W2_0, b2_0, eps_0, gamma_0, beta_0,
           W1_1, b1_1, W2_1, b2_1, eps_1, gamma_1, beta_1,
           W1_2, b1_2, W2_2, b2_2, eps_2, gamma_2, beta_2):
    npad_e = _EPAD - _E
    src_pad = jnp.arange(npad_e, dtype=jnp.int32) % _N
    dst_pad = (_N + (jnp.arange(npad_e, dtype=jnp.int32) % (_NPAD - _N)))
    src1 = jnp.concatenate([edge_index[0], src_pad])
    dst1 = jnp.concatenate([edge_index[1], dst_pad])
    zrows = jnp.zeros((_RPS, _D), jnp.float32)

    layers = [
        (W1_0, b1_0, W2_0, b2_0, eps_0, gamma_0, beta_0),
        (W1_1, b1_1, W2_1, b2_1, eps_1, gamma_1, beta_1),
        (W1_2, b1_2, W2_2, b2_2, eps_2, gamma_2, beta_2),
    ]
    h = x
    for i, (W1, b1, W2, b2, eps, gamma, beta) in enumerate(layers):
        agg = _sc_segment_sum(h, src1, dst1, zrows)
        gs = gamma * _BN_SCALE
        W2f = W2 * gs[None, :]
        b2f = (b2 * gs + beta).reshape(1, _D)
        epsv = jnp.broadcast_to(1.0 + eps, (1, _D)).astype(jnp.float32)
        h = _mlp(h, agg, W1, b1.reshape(1, _D), W2f, b2f, epsv, i < 2)
    return h


# speedup vs baseline: 1.6705x; 1.1829x over previous
import functools

import jax
import jax.numpy as jnp
import numpy as np
from jax import lax
from jax.experimental import pallas as pl
from jax.experimental.pallas import tpu as pltpu
from jax.experimental.pallas import tpu_sc as plsc

_N = 10000
_D = 128
_E = 320000
_BN_EPS = 1e-5
_BN_SCALE = float(1.0 / np.sqrt(1.0 + _BN_EPS))

_NC = 2
_NS = 16
_NW = _NC * _NS
_CH = 120
_CPT = 84
_EPAD = _NW * _CPT * _CH
_NPAD = 10112
_RPS = _NPAD // _NS
_RLAST = _N - (_NS - 1) * _RPS


def _sc_segment_sum(h, src1, dst1, zrows):
    mesh = plsc.VectorSubcoreMesh(
        core_axis_name="c", subcore_axis_name="s",
        num_cores=_NC, num_subcores=_NS)

    @functools.partial(
        pl.kernel,
        out_type=jax.ShapeDtypeStruct((_NC, _N, _D), jnp.float32),
        mesh=mesh,
        scratch_types=[
            pltpu.VMEM_SHARED((_NPAD, _D), jnp.float32),
        ] + [pltpu.VMEM((_CH,), jnp.int32) for _ in range(12)]
          + [pltpu.VMEM((_CH, _D), jnp.float32) for _ in range(3)]
          + [pltpu.SemaphoreType.DMA for _ in range(9)],
    )
    def k(h_hbm, src_hbm, dst_hbm, z_hbm, out_hbm, agg_sh,
          si0, si1, si2, si3, si4, si5, di0, di1, di2, di3, di4, di5,
          r0, r1, r2, mi0, mi1, mi2, mi3, mi4, mi5, mg0, mg1, mg2):
        sbuf = (si0, si1, si2, si3, si4, si5)
        dbuf = (di0, di1, di2, di3, di4, di5)
        rows = (r0, r1, r2)
        mi = (mi0, mi1, mi2, mi3, mi4, mi5)
        mg = (mg0, mg1, mg2)
        c = lax.axis_index("c")
        s = lax.axis_index("s")
        wid = s * _NC + c
        rbase = s * _RPS
        ebase = wid * (_CPT * _CH)

        pltpu.sync_copy(z_hbm, agg_sh.at[pl.ds(rbase, _RPS)])
        plsc.subcore_barrier()

        @pl.loop(0, _CPT // 6)
        def _(t):
            base = ebase + t * (6 * _CH)
            ia = []
            ib = []
            for kk in range(6):
                off = base + kk * _CH
                ia.append(pltpu.async_copy(src_hbm.at[pl.ds(off, _CH)],
                                           sbuf[kk], mi[kk]))
                ib.append(pltpu.async_copy(dst_hbm.at[pl.ds(off, _CH)],
                                           dbuf[kk], mi[kk]))
            g = [None] * 6
            sc = [None] * 6
            for kk in range(3):
                ia[kk].wait()
                ib[kk].wait()
                g[kk] = pltpu.async_copy(h_hbm.at[sbuf[kk]], rows[kk], mg[kk])
            for kk in range(3):
                g[kk].wait()
                sc[kk] = pltpu.async_copy(rows[kk], agg_sh.at[dbuf[kk]],
                                          mg[kk], add=True)
            for kk in range(3, 6):
                b = kk - 3
                sc[b].wait()
                ia[kk].wait()
                ib[kk].wait()
                g[kk] = pltpu.async_copy(h_hbm.at[sbuf[kk]], rows[b], mg[b])
            for kk in range(3, 6):
                b = kk - 3
                g[kk].wait()
                sc[kk] = pltpu.async_copy(rows[b], agg_sh.at[dbuf[kk]],
                                          mg[b], add=True)
            for kk in range(3, 6):
                sc[kk].wait()

        plsc.subcore_barrier()

        @pl.when(s < _NS - 1)
        def _():
            pltpu.sync_copy(agg_sh.at[pl.ds(rbase, _RPS)],
                            out_hbm.at[c, pl.ds(rbase, _RPS)])

        @pl.when(s == _NS - 1)
        def _():
            pltpu.sync_copy(agg_sh.at[pl.ds(rbase, _RLAST)],
                            out_hbm.at[c, pl.ds(rbase, _RLAST)])

    return k(h, src1, dst1, zrows)


def _mlp(h, agg, W1, b1r, W2f, b2f, epsv, relu_out):
    BR = 1000

    def body(eps_ref, h_ref, agg_ref, w1_ref, b1_ref, w2_ref, b2_ref, out_ref):
        z = eps_ref[...] * h_ref[...] + agg_ref[0] + agg_ref[1]
        z = jnp.dot(z, w1_ref[...], preferred_element_type=jnp.float32) + b1_ref[...]
        z = jnp.maximum(z, 0.0)
        z = jnp.dot(z, w2_ref[...], preferred_element_type=jnp.float32) + b2_ref[...]
        if relu_out:
            z = jnp.maximum(z, 0.0)
        out_ref[...] = z

    return pl.pallas_call(
        body,
        grid=(_N // BR,),
        in_specs=[
            pl.BlockSpec((1, _D), lambda i: (0, 0)),
            pl.BlockSpec((BR, _D), lambda i: (i, 0)),
            pl.BlockSpec((_NC, BR, _D), lambda i: (0, i, 0)),
            pl.BlockSpec((_D, _D), lambda i: (0, 0)),
            pl.BlockSpec((1, _D), lambda i: (0, 0)),
            pl.BlockSpec((_D, _D), lambda i: (0, 0)),
            pl.BlockSpec((1, _D), lambda i: (0, 0)),
        ],
        out_specs=pl.BlockSpec((BR, _D), lambda i: (i, 0)),
        out_shape=jax.ShapeDtypeStruct((_N, _D), jnp.float32),
    )(epsv, h, agg, W1, b1r, W2f, b2f)


def kernel(x, edge_index,
           W1_0, b1_0, W2_0, b2_0, eps_0, gamma_0, beta_0,
           W1_1, b1_1, W2_1, b2_1, eps_1, gamma_1, beta_1,
           W1_2, b1_2, W2_2, b2_2, eps_2, gamma_2, beta_2):
    npad_e = _EPAD - _E
    src_pad = jnp.arange(npad_e, dtype=jnp.int32) % _N
    dst_pad = (_N + (jnp.arange(npad_e, dtype=jnp.int32) % (_NPAD - _N)))
    src1 = jnp.concatenate([edge_index[0], src_pad])
    dst1 = jnp.concatenate([edge_index[1], dst_pad])
    zrows = jnp.zeros((_RPS, _D), jnp.float32)

    layers = [
        (W1_0, b1_0, W2_0, b2_0, eps_0, gamma_0, beta_0),
        (W1_1, b1_1, W2_1, b2_1, eps_1, gamma_1, beta_1),
        (W1_2, b1_2, W2_2, b2_2, eps_2, gamma_2, beta_2),
    ]
    h = x
    for i, (W1, b1, W2, b2, eps, gamma, beta) in enumerate(layers):
        agg = _sc_segment_sum(h, src1, dst1, zrows)
        gs = gamma * _BN_SCALE
        W2f = W2 * gs[None, :]
        b2f = (b2 * gs + beta).reshape(1, _D)
        epsv = jnp.broadcast_to(1.0 + eps, (1, _D)).astype(jnp.float32)
        h = _mlp(h, agg, W1, b1.reshape(1, _D), W2f, b2f, epsv, i < 2)
    return h
